# Initial kernel scaffold; baseline (speedup 1.0000x reference)
import jax
import jax.numpy as jnp
from jax.experimental import pallas as pl


def kernel(x, SubjId, W1, b1, W2, b2):
    raise NotImplementedError("write your pallas kernel here")


# speedup vs baseline: 7.8714x; 7.8714x over previous
import functools

import jax
import jax.numpy as jnp
from jax import lax
from jax.experimental import pallas as pl
from jax.experimental.pallas import tpu as pltpu
from jax.experimental.pallas import tpu_sc as plsc

_NUM_PART = 21
_D_IN = 128
_D_HID = 16
_B = 16384
_NE = 24
_H_ALL = _NE * _D_HID
_BLK = 2048

_NC = 2
_NS = 16
_L = 16
_NW = _NC * _NS
_CH = _B // _NW


def _heads_body(x_ref, w1t_ref, b1_ref, w2_ref, sel_ref, b2_ref, out_ref):
    h = jnp.dot(x_ref[...], w1t_ref[...], preferred_element_type=jnp.float32)
    h = jnp.maximum(h + b1_ref[...], 0.0)
    t = h * w2_ref[...]
    o = jnp.dot(t, sel_ref[...], preferred_element_type=jnp.float32)
    out_ref[...] = jnp.maximum(o + b2_ref[...], 0.0)


def _compute_all_heads(x, w1t, b1f, w2f, sel, b2f):
    return pl.pallas_call(
        _heads_body,
        grid=(_B // _BLK,),
        in_specs=[
            pl.BlockSpec((_BLK, _D_IN), lambda i: (i, 0)),
            pl.BlockSpec((_D_IN, _H_ALL), lambda i: (0, 0)),
            pl.BlockSpec((1, _H_ALL), lambda i: (0, 0)),
            pl.BlockSpec((1, _H_ALL), lambda i: (0, 0)),
            pl.BlockSpec((_H_ALL, _NE), lambda i: (0, 0)),
            pl.BlockSpec((1, _NE), lambda i: (0, 0)),
        ],
        out_specs=pl.BlockSpec((_BLK, _NE), lambda i: (i, 0)),
        out_shape=jax.ShapeDtypeStruct((_B, _NE), jnp.float32),
    )(x, w1t, b1f, w2f, sel, b2f)


_sc_mesh = plsc.VectorSubcoreMesh(core_axis_name="c", subcore_axis_name="s")


@functools.partial(
    pl.kernel,
    mesh=_sc_mesh,
    out_type=jax.ShapeDtypeStruct((_B,), jnp.float32),
    scratch_types=[
        pltpu.VMEM((_CH,), jnp.int32),
        pltpu.VMEM((_CH * _NE,), jnp.float32),
        pltpu.VMEM((_CH,), jnp.float32),
    ],
    compiler_params=pltpu.CompilerParams(needs_layout_passes=False),
)
def _sc_select(oall_hbm, subj_hbm, out_hbm, subj_v, tab_v, out_v):
    wid = lax.axis_index("s") * _NC + lax.axis_index("c")
    base = wid * _CH
    pltpu.sync_copy(subj_hbm.at[pl.ds(base, _CH)], subj_v)
    pltpu.sync_copy(oall_hbm.at[pl.ds(base * _NE, _CH * _NE)], tab_v)

    def body(j, carry):
        off = j * _L
        subj = subj_v[pl.ds(off, _L)]
        idx = (lax.iota(jnp.int32, _L) + off) * _NE + subj
        out_v[pl.ds(off, _L)] = plsc.load_gather(tab_v, [idx])
        return carry

    lax.fori_loop(0, _CH // _L, body, 0)
    pltpu.sync_copy(out_v, out_hbm.at[pl.ds(base, _CH)])


def kernel(x, SubjId, W1, b1, W2, b2):
    pad = _NE - _NUM_PART
    w1t = jnp.pad(W1, ((0, pad), (0, 0), (0, 0))).reshape(_H_ALL, _D_IN).T
    b1f = jnp.pad(b1, ((0, pad), (0, 0))).reshape(1, _H_ALL)
    w2f = jnp.pad(W2, ((0, pad), (0, 0), (0, 0))).reshape(1, _H_ALL)
    b2f = jnp.pad(b2, ((0, pad), (0, 0))).reshape(1, _NE)
    sel = (jnp.arange(_H_ALL, dtype=jnp.int32)[:, None] // _D_HID
           == jnp.arange(_NE, dtype=jnp.int32)[None, :]).astype(jnp.float32)
    out_all = _compute_all_heads(x, w1t, b1f, w2f, sel, b2f)
    out = _sc_select(out_all.reshape(_B * _NE), SubjId)
    return out.reshape(_B, 1)


# speedup vs baseline: 9.8346x; 1.2494x over previous
import functools

import jax
import jax.numpy as jnp
from jax import lax
from jax.experimental import pallas as pl
from jax.experimental.pallas import tpu as pltpu
from jax.experimental.pallas import tpu_sc as plsc

_NUM_PART = 21
_D_IN = 128
_D_HID = 16
_B = 16384
_H_ALL = _NUM_PART * _D_HID
_BLK = 2048

_NC = 2
_NS = 16
_L = 16
_NW = _NC * _NS
_CH = _B // _NW


def _heads_body(x_ref, w1_ref, b1_ref, w2_ref, sel_ref, b2_ref, out_ref):
    h = lax.dot_general(x_ref[...], w1_ref[...],
                        dimension_numbers=(((1,), (1,)), ((), ())),
                        preferred_element_type=jnp.float32)
    h = jnp.maximum(h + b1_ref[...], 0.0)
    t = h * w2_ref[...]
    o = jnp.dot(t, sel_ref[...], preferred_element_type=jnp.float32)
    out_ref[...] = jnp.maximum(o + b2_ref[...], 0.0)


def _compute_all_heads(x, w1f, b1f, w2f, sel, b2f):
    return pl.pallas_call(
        _heads_body,
        grid=(_B // _BLK,),
        in_specs=[
            pl.BlockSpec((_BLK, _D_IN), lambda i: (i, 0)),
            pl.BlockSpec((_H_ALL, _D_IN), lambda i: (0, 0)),
            pl.BlockSpec((1, _H_ALL), lambda i: (0, 0)),
            pl.BlockSpec((1, _H_ALL), lambda i: (0, 0)),
            pl.BlockSpec((_H_ALL, _NUM_PART), lambda i: (0, 0)),
            pl.BlockSpec((1, _NUM_PART), lambda i: (0, 0)),
        ],
        out_specs=pl.BlockSpec((_BLK, _NUM_PART), lambda i: (i, 0)),
        out_shape=jax.ShapeDtypeStruct((_B, _NUM_PART), jnp.float32),
    )(x, w1f, b1f, w2f, sel, b2f)


_sc_mesh = plsc.VectorSubcoreMesh(core_axis_name="c", subcore_axis_name="s")


@functools.partial(
    pl.kernel,
    mesh=_sc_mesh,
    out_type=jax.ShapeDtypeStruct((_B,), jnp.float32),
    scratch_types=[
        pltpu.VMEM((_CH,), jnp.int32),
        pltpu.VMEM((_CH, _NUM_PART), jnp.float32),
        pltpu.VMEM((_CH,), jnp.float32),
    ],
    compiler_params=pltpu.CompilerParams(needs_layout_passes=False),
)
def _sc_select(oall_hbm, subj_hbm, out_hbm, subj_v, tab_v, out_v):
    wid = lax.axis_index("s") * _NC + lax.axis_index("c")
    base = wid * _CH
    pltpu.sync_copy(subj_hbm.at[pl.ds(base, _CH)], subj_v)
    pltpu.sync_copy(oall_hbm.at[pl.ds(base, _CH)], tab_v)

    def body(j, carry):
        off = j * _L
        subj = subj_v[pl.ds(off, _L)]
        rows = lax.iota(jnp.int32, _L) + off
        out_v[pl.ds(off, _L)] = plsc.load_gather(tab_v, [rows, subj])
        return carry

    lax.fori_loop(0, _CH // _L, body, 0)
    pltpu.sync_copy(out_v, out_hbm.at[pl.ds(base, _CH)])


def kernel(x, SubjId, W1, b1, W2, b2):
    w1f = W1.reshape(_H_ALL, _D_IN)
    b1f = b1.reshape(1, _H_ALL)
    w2f = W2.reshape(1, _H_ALL)
    b2f = b2.reshape(1, _NUM_PART)
    sel = (jnp.arange(_H_ALL, dtype=jnp.int32)[:, None] // _D_HID
           == jnp.arange(_NUM_PART, dtype=jnp.int32)[None, :]).astype(jnp.float32)
    out_all = _compute_all_heads(x, w1f, b1f, w2f, sel, b2f)
    out = _sc_select(out_all, SubjId)
    return out.reshape(_B, 1)
